# Initial kernel scaffold; baseline (speedup 1.0000x reference)
import jax
import jax.numpy as jnp
from jax.experimental import pallas as pl


def kernel(sites, bonds, indices1, indices2, W_sig, b_sig, W_soft, b_soft):
    raise NotImplementedError("write your pallas kernel here")


# speedup vs baseline: 1.0812x; 1.0812x over previous
import functools

import jax
import jax.numpy as jnp
from jax import lax
from jax.experimental import pallas as pl
from jax.experimental.pallas import tpu as pltpu
from jax.experimental.pallas import tpu_sc as plsc

N = 10000
E = 320000
D = 128
DB = 16
DD = 2 * D

NC = 2
NS = 16
NW = NC * NS
EPW = E // NW
CH = 40
TPC = EPW // CH
NPAD = 10240
RPS = NPAD // NS


def _tables_body(s_ref, wt_ref, wu_ref, t_ref, u_ref):
    s = s_ref[...]
    dn = (((1,), (1,)), ((), ()))
    t_ref[...] = lax.dot_general(s, wt_ref[...], dn,
                                 preferred_element_type=jnp.float32)
    u_ref[...] = lax.dot_general(s, wu_ref[...], dn,
                                 preferred_element_type=jnp.float32)


def _make_tables(sites2d, w_t, w_u):
    bn = 1000
    return pl.pallas_call(
        _tables_body,
        grid=(N // bn,),
        in_specs=[
            pl.BlockSpec((bn, D), lambda i: (i, 0)),
            pl.BlockSpec((DD, D), lambda i: (0, 0)),
            pl.BlockSpec((DD, D), lambda i: (0, 0)),
        ],
        out_specs=[
            pl.BlockSpec((bn, DD), lambda i: (i, 0)),
            pl.BlockSpec((bn, DD), lambda i: (i, 0)),
        ],
        out_shape=[
            jax.ShapeDtypeStruct((N, DD), jnp.float32),
            jax.ShapeDtypeStruct((N, DD), jnp.float32),
        ],
    )(sites2d, w_t, w_u)


def _bond_body(b_ref, wb_ref, bias_ref, bb_ref):
    dn = (((1,), (1,)), ((), ()))
    bb_ref[...] = lax.dot_general(b_ref[...], wb_ref[...], dn,
                                  preferred_element_type=jnp.float32) + bias_ref[...]


def _make_bond(bonds2d, w_b, bias):
    be = 4000
    return pl.pallas_call(
        _bond_body,
        grid=(E // be,),
        in_specs=[
            pl.BlockSpec((be, DB), lambda i: (i, 0)),
            pl.BlockSpec((DD, DB), lambda i: (0, 0)),
            pl.BlockSpec((1, DD), lambda i: (0, 0)),
        ],
        out_specs=pl.BlockSpec((be, DD), lambda i: (i, 0)),
        out_shape=jax.ShapeDtypeStruct((E, DD), jnp.float32),
    )(bonds2d, w_b, bias)


def _sc_body(t_hbm, u_hbm, bb_hbm, i1_hbm, i2_hbm, z_hbm, out_hbm,
             i1v, i2v, tv, uv, bbv, mv, agg, sem1, sem2):
    cid = lax.axis_index("c")
    sid = lax.axis_index("s")
    wid = sid * NC + cid

    row0 = sid * RPS
    pltpu.sync_copy(z_hbm, agg.at[pl.ds(row0, RPS)])
    plsc.subcore_barrier()

    def step(t, carry):
        base = pl.multiple_of(wid * EPW + t * CH, 8)
        pltpu.sync_copy(i1_hbm.at[pl.ds(base, CH)], i1v)
        pltpu.sync_copy(i2_hbm.at[pl.ds(base, CH)], i2v)
        cp1 = pltpu.async_copy(t_hbm.at[i1v], tv, sem1)
        cp2 = pltpu.async_copy(u_hbm.at[i2v], uv, sem2)
        pltpu.sync_copy(bb_hbm.at[pl.ds(base, CH)], bbv)
        cp1.wait()
        cp2.wait()

        def erow(e, c2):
            for k in range(D // 16):
                sl = pl.ds(k * 16, 16)
                sh = pl.ds(D + k * 16, 16)
                ls = tv[e, sl] + uv[e, sl] + bbv[e, sl]
                lc = tv[e, sh] + uv[e, sh] + bbv[e, sh]
                gate = 1.0 / (1.0 + jnp.exp(-ls))
                mv[e, sl] = gate * jnp.maximum(lc, 0.0)
            return c2

        lax.fori_loop(0, CH, erow, 0)
        pltpu.sync_copy(mv, agg.at[i1v], add=True)
        return carry

    lax.fori_loop(0, TPC, step, 0)
    plsc.subcore_barrier()
    pltpu.sync_copy(agg.at[pl.ds(row0, RPS)],
                    out_hbm.at[cid, pl.ds(row0, RPS)])


def _make_sc(t_tab, u_tab, bb, i1, i2, zrows):
    mesh = plsc.VectorSubcoreMesh(core_axis_name="c", subcore_axis_name="s")
    k = functools.partial(
        pl.kernel,
        mesh=mesh,
        out_type=jax.ShapeDtypeStruct((NC, NPAD, D), jnp.float32),
        scratch_types=[
            pltpu.VMEM((CH,), jnp.int32),
            pltpu.VMEM((CH,), jnp.int32),
            pltpu.VMEM((CH, DD), jnp.float32),
            pltpu.VMEM((CH, DD), jnp.float32),
            pltpu.VMEM((CH, DD), jnp.float32),
            pltpu.VMEM((CH, D), jnp.float32),
            pltpu.VMEM_SHARED((NPAD, D), jnp.float32),
            pltpu.SemaphoreType.DMA,
            pltpu.SemaphoreType.DMA,
        ],
    )(_sc_body)
    return k(t_tab, u_tab, bb, i1, i2, zrows)


def _combine_body(s_ref, a_ref, o_ref):
    a = a_ref[...]
    o_ref[...] = s_ref[...] + (a[0] + a[1])[None]


def _make_combine(sites, agg):
    bn = 1000
    return pl.pallas_call(
        _combine_body,
        grid=(N // bn,),
        in_specs=[
            pl.BlockSpec((1, bn, D), lambda i: (0, i, 0)),
            pl.BlockSpec((NC, bn, D), lambda i: (0, i, 0)),
        ],
        out_specs=pl.BlockSpec((1, bn, D), lambda i: (0, i, 0)),
        out_shape=jax.ShapeDtypeStruct((1, N, D), jnp.float32),
    )(sites, agg)


def kernel(sites, bonds, indices1, indices2, W_sig, b_sig, W_soft, b_soft):
    sites2d = sites[0]
    bonds2d = bonds[0]
    w_t = jnp.concatenate([W_sig[:, :D], W_soft[:, :D]], axis=0)
    w_u = jnp.concatenate([W_sig[:, D:2 * D], W_soft[:, D:2 * D]], axis=0)
    w_b = jnp.concatenate([W_sig[:, 2 * D:], W_soft[:, 2 * D:]], axis=0)
    bias = jnp.concatenate([b_sig, b_soft])[None, :]
    zrows = jnp.zeros((RPS, D), jnp.float32)

    t_tab, u_tab = _make_tables(sites2d, w_t, w_u)
    bb = _make_bond(bonds2d, w_b, bias)
    agg = _make_sc(t_tab, u_tab, bb, indices1, indices2, zrows)
    return _make_combine(sites, agg)


# speedup vs baseline: 2.1293x; 1.9693x over previous
import functools

import jax
import jax.numpy as jnp
from jax import lax
from jax.experimental import pallas as pl
from jax.experimental.pallas import tpu as pltpu
from jax.experimental.pallas import tpu_sc as plsc

N = 10000
E = 320000
D = 128
DB = 16
DD = 2 * D

NC = 2
NS = 16
NW = NC * NS
EPW = E // NW
CH = 16
TPC = EPW // CH
NPAD = 10240
RPS = NPAD // NS


def _tables_body(s_ref, wt_ref, wu_ref, t_ref, u_ref):
    s = s_ref[...]
    dn = (((1,), (1,)), ((), ()))
    t_ref[...] = lax.dot_general(s, wt_ref[...], dn,
                                 preferred_element_type=jnp.float32)
    u_ref[...] = lax.dot_general(s, wu_ref[...], dn,
                                 preferred_element_type=jnp.float32)


def _make_tables(sites2d, w_t, w_u):
    bn = 1000
    return pl.pallas_call(
        _tables_body,
        grid=(N // bn,),
        in_specs=[
            pl.BlockSpec((bn, D), lambda i: (i, 0)),
            pl.BlockSpec((DD, D), lambda i: (0, 0)),
            pl.BlockSpec((DD, D), lambda i: (0, 0)),
        ],
        out_specs=[
            pl.BlockSpec((bn, DD), lambda i: (i, 0)),
            pl.BlockSpec((bn, DD), lambda i: (i, 0)),
        ],
        out_shape=[
            jax.ShapeDtypeStruct((N, DD), jnp.float32),
            jax.ShapeDtypeStruct((N, DD), jnp.float32),
        ],
    )(sites2d, w_t, w_u)


def _bond_body(b_ref, wb_ref, bias_ref, bb_ref):
    dn = (((1,), (1,)), ((), ()))
    bb_ref[...] = lax.dot_general(b_ref[...], wb_ref[...], dn,
                                  preferred_element_type=jnp.float32) + bias_ref[...]


def _make_bond(bonds2d, w_b, bias):
    be = 4000
    return pl.pallas_call(
        _bond_body,
        grid=(E // be,),
        in_specs=[
            pl.BlockSpec((be, DB), lambda i: (i, 0)),
            pl.BlockSpec((DD, DB), lambda i: (0, 0)),
            pl.BlockSpec((1, DD), lambda i: (0, 0)),
        ],
        out_specs=pl.BlockSpec((be, DD), lambda i: (i, 0)),
        out_shape=jax.ShapeDtypeStruct((E, DD), jnp.float32),
    )(bonds2d, w_b, bias)


def _sc_body(t_hbm, u_hbm, bb_hbm, i1_hbm, i2_hbm, z_hbm, out_hbm,
             i1v0, i1v1, i2v0, i2v1, tv0, tv1, uv0, uv1, bbv0, bbv1, mv,
             agg, semi0, semi1, semd0, semd1):
    cid = lax.axis_index("c")
    sid = lax.axis_index("s")
    wid = sid * NC + cid
    i1v = [i1v0, i1v1]
    i2v = [i2v0, i2v1]
    tv = [tv0, tv1]
    uv = [uv0, uv1]
    bbv = [bbv0, bbv1]
    semi = [semi0, semi1]
    semd = [semd0, semd1]

    def cbase(t):
        t = jnp.minimum(t, TPC - 1)
        return pl.multiple_of(wid * EPW + t * CH, 8)

    def issue_idx(p, t):
        base = cbase(t)
        pltpu.async_copy(i1_hbm.at[pl.ds(base, CH)], i1v[p], semi[p])
        pltpu.async_copy(i2_hbm.at[pl.ds(base, CH)], i2v[p], semi[p])

    def wait_idx(p):
        pltpu.make_async_copy(i1_hbm.at[pl.ds(0, CH)], i1v[p], semi[p]).wait()
        pltpu.make_async_copy(i2_hbm.at[pl.ds(0, CH)], i2v[p], semi[p]).wait()

    def issue_dat(p, t):
        base = cbase(t)
        pltpu.async_copy(t_hbm.at[i1v[p]], tv[p], semd[p])
        pltpu.async_copy(u_hbm.at[i2v[p]], uv[p], semd[p])
        pltpu.async_copy(bb_hbm.at[pl.ds(base, CH)], bbv[p], semd[p])

    def wait_dat(p):
        pltpu.make_async_copy(t_hbm.at[pl.ds(0, CH)], tv[p], semd[p]).wait()
        pltpu.make_async_copy(u_hbm.at[pl.ds(0, CH)], uv[p], semd[p]).wait()
        pltpu.make_async_copy(bb_hbm.at[pl.ds(0, CH)], bbv[p], semd[p]).wait()

    def compute(p):
        tp, up, bp = tv[p], uv[p], bbv[p]

        def erow(e, c2):
            nk = D // 16
            es = []
            for k in range(nk):
                sl = pl.ds(k * 16, 16)
                es.append(jnp.exp(-(tp[e, sl] + up[e, sl] + bp[e, sl])))
            gs = [1.0 / (1.0 + x) for x in es]
            for k in range(nk):
                sh = pl.ds(D + k * 16, 16)
                lc = tp[e, sh] + up[e, sh] + bp[e, sh]
                mv[e, pl.ds(k * 16, 16)] = gs[k] * jnp.maximum(lc, 0.0)
            return c2

        lax.fori_loop(0, CH, erow, 0)

    row0 = sid * RPS
    pltpu.sync_copy(z_hbm, agg.at[pl.ds(row0, RPS)])
    plsc.subcore_barrier()

    issue_idx(0, 0)
    wait_idx(0)
    issue_dat(0, 0)
    issue_idx(1, 1)

    def pair(tt, carry):
        t0 = 2 * tt
        wait_idx(1)
        issue_dat(1, t0 + 1)
        wait_dat(0)
        compute(0)
        pltpu.sync_copy(mv, agg.at[i1v[0]], add=True)
        issue_idx(0, t0 + 2)
        wait_idx(0)
        issue_dat(0, t0 + 2)
        wait_dat(1)
        compute(1)
        pltpu.sync_copy(mv, agg.at[i1v[1]], add=True)
        issue_idx(1, t0 + 3)
        return carry

    lax.fori_loop(0, TPC // 2, pair, 0)
    wait_idx(1)
    wait_dat(0)
    compute(0)
    pltpu.sync_copy(mv, agg.at[i1v[0]], add=True)

    plsc.subcore_barrier()
    pltpu.sync_copy(agg.at[pl.ds(row0, RPS)],
                    out_hbm.at[cid, pl.ds(row0, RPS)])


def _make_sc(t_tab, u_tab, bb, i1, i2, zrows):
    mesh = plsc.VectorSubcoreMesh(core_axis_name="c", subcore_axis_name="s")
    k = functools.partial(
        pl.kernel,
        mesh=mesh,
        out_type=jax.ShapeDtypeStruct((NC, NPAD, D), jnp.float32),
        scratch_types=[
            pltpu.VMEM((CH,), jnp.int32),
            pltpu.VMEM((CH,), jnp.int32),
            pltpu.VMEM((CH,), jnp.int32),
            pltpu.VMEM((CH,), jnp.int32),
            pltpu.VMEM((CH, DD), jnp.float32),
            pltpu.VMEM((CH, DD), jnp.float32),
            pltpu.VMEM((CH, DD), jnp.float32),
            pltpu.VMEM((CH, DD), jnp.float32),
            pltpu.VMEM((CH, DD), jnp.float32),
            pltpu.VMEM((CH, DD), jnp.float32),
            pltpu.VMEM((CH, D), jnp.float32),
            pltpu.VMEM_SHARED((NPAD, D), jnp.float32),
            pltpu.SemaphoreType.DMA,
            pltpu.SemaphoreType.DMA,
            pltpu.SemaphoreType.DMA,
            pltpu.SemaphoreType.DMA,
        ],
    )(_sc_body)
    return k(t_tab, u_tab, bb, i1, i2, zrows)


def _combine_body(s_ref, a_ref, o_ref):
    a = a_ref[...]
    o_ref[...] = s_ref[...] + (a[0] + a[1])[None]


def _make_combine(sites, agg):
    bn = 1000
    return pl.pallas_call(
        _combine_body,
        grid=(N // bn,),
        in_specs=[
            pl.BlockSpec((1, bn, D), lambda i: (0, i, 0)),
            pl.BlockSpec((NC, bn, D), lambda i: (0, i, 0)),
        ],
        out_specs=pl.BlockSpec((1, bn, D), lambda i: (0, i, 0)),
        out_shape=jax.ShapeDtypeStruct((1, N, D), jnp.float32),
    )(sites, agg)


def kernel(sites, bonds, indices1, indices2, W_sig, b_sig, W_soft, b_soft):
    sites2d = sites[0]
    bonds2d = bonds[0]
    w_t = jnp.concatenate([W_sig[:, :D], W_soft[:, :D]], axis=0)
    w_u = jnp.concatenate([W_sig[:, D:2 * D], W_soft[:, D:2 * D]], axis=0)
    w_b = jnp.concatenate([W_sig[:, 2 * D:], W_soft[:, 2 * D:]], axis=0)
    bias = jnp.concatenate([b_sig, b_soft])[None, :]
    zrows = jnp.zeros((RPS, D), jnp.float32)

    t_tab, u_tab = _make_tables(sites2d, w_t, w_u)
    bb = _make_bond(bonds2d, w_b, bias)
    agg = _make_sc(t_tab, u_tab, bb, indices1, indices2, zrows)
    return _make_combine(sites, agg)


# speedup vs baseline: 2.2618x; 1.0622x over previous
import functools

import jax
import jax.numpy as jnp
from jax import lax
from jax.experimental import pallas as pl
from jax.experimental.pallas import tpu as pltpu
from jax.experimental.pallas import tpu_sc as plsc

N = 10000
E = 320000
D = 128
DB = 16
DD = 2 * D

NC = 2
NS = 16
NW = NC * NS
EPW = E // NW
CH = 16
TPC = EPW // CH
NPAD = 10240
RPS = NPAD // NS


def _tables_body(s_ref, wt_ref, wu_ref, t_ref, u_ref):
    s = s_ref[...]
    dn = (((1,), (1,)), ((), ()))
    t_ref[...] = lax.dot_general(s, wt_ref[...], dn,
                                 preferred_element_type=jnp.float32)
    u_ref[...] = lax.dot_general(s, wu_ref[...], dn,
                                 preferred_element_type=jnp.float32)


def _make_tables(sites2d, w_t, w_u):
    bn = 1000
    return pl.pallas_call(
        _tables_body,
        grid=(N // bn,),
        in_specs=[
            pl.BlockSpec((bn, D), lambda i: (i, 0)),
            pl.BlockSpec((DD, D), lambda i: (0, 0)),
            pl.BlockSpec((DD, D), lambda i: (0, 0)),
        ],
        out_specs=[
            pl.BlockSpec((bn, DD), lambda i: (i, 0)),
            pl.BlockSpec((bn, DD), lambda i: (i, 0)),
        ],
        out_shape=[
            jax.ShapeDtypeStruct((N, DD), jnp.float32),
            jax.ShapeDtypeStruct((N, DD), jnp.float32),
        ],
    )(sites2d, w_t, w_u)


def _bond_body(b_ref, wb_ref, bias_ref, bb_ref):
    dn = (((1,), (1,)), ((), ()))
    bb_ref[...] = lax.dot_general(b_ref[...], wb_ref[...], dn,
                                  preferred_element_type=jnp.float32) + bias_ref[...]


def _make_bond(bonds2d, w_b, bias):
    be = 4000
    return pl.pallas_call(
        _bond_body,
        grid=(E // be,),
        in_specs=[
            pl.BlockSpec((be, DB), lambda i: (i, 0)),
            pl.BlockSpec((DD, DB), lambda i: (0, 0)),
            pl.BlockSpec((1, DD), lambda i: (0, 0)),
        ],
        out_specs=pl.BlockSpec((be, DD), lambda i: (i, 0)),
        out_shape=jax.ShapeDtypeStruct((E, DD), jnp.float32),
    )(bonds2d, w_b, bias)


def _sc_body(t_hbm, u_hbm, bb_hbm, i1_hbm, i2_hbm, z_hbm, out_hbm,
             i1v0, i1v1, i1v2, i2v0, i2v1, i2v2, tv0, tv1, tv2,
             uv0, uv1, uv2, bbv0, bbv1, bbv2, mv0, mv1, mv2,
             sv0, sv1, sv2, agg,
             semi0, semi1, semi2, semd0, semd1, semd2,
             sems0, sems1, sems2):
    cid = lax.axis_index("c")
    sid = lax.axis_index("s")
    wid = sid * NC + cid
    i1v = [i1v0, i1v1, i1v2]
    i2v = [i2v0, i2v1, i2v2]
    tv = [tv0, tv1, tv2]
    uv = [uv0, uv1, uv2]
    bbv = [bbv0, bbv1, bbv2]
    mv = [mv0, mv1, mv2]
    sv = [sv0, sv1, sv2]
    semi = [semi0, semi1, semi2]
    semd = [semd0, semd1, semd2]
    sems = [sems0, sems1, sems2]

    def cbase(t):
        t = jnp.minimum(t, TPC - 1)
        return pl.multiple_of(wid * EPW + t * CH, 8)

    def issue_idx(p, t):
        base = cbase(t)
        pltpu.async_copy(i1_hbm.at[pl.ds(base, CH)], i1v[p], semi[p])
        pltpu.async_copy(i2_hbm.at[pl.ds(base, CH)], i2v[p], semi[p])

    def wait_idx(p):
        pltpu.make_async_copy(i1_hbm.at[pl.ds(0, CH)], i1v[p], semi[p]).wait()
        pltpu.make_async_copy(i2_hbm.at[pl.ds(0, CH)], i2v[p], semi[p]).wait()

    def issue_dat(p, t):
        base = cbase(t)
        pltpu.async_copy(t_hbm.at[i1v[p]], tv[p], semd[p])
        pltpu.async_copy(u_hbm.at[i2v[p]], uv[p], semd[p])
        pltpu.async_copy(bb_hbm.at[pl.ds(base, CH)], bbv[p], semd[p])

    def wait_dat(p):
        pltpu.make_async_copy(t_hbm.at[pl.ds(0, CH)], tv[p], semd[p]).wait()
        pltpu.make_async_copy(u_hbm.at[pl.ds(0, CH)], uv[p], semd[p]).wait()
        pltpu.make_async_copy(bb_hbm.at[pl.ds(0, CH)], bbv[p], semd[p]).wait()

    def compute(p):
        tp, up, bp = tv[p], uv[p], bbv[p]
        mp = mv[p]

        def erow(e, c2):
            nk = D // 16
            es = []
            for k in range(nk):
                sl = pl.ds(k * 16, 16)
                es.append(jnp.exp(-(tp[e, sl] + up[e, sl] + bp[e, sl])))
            gs = [1.0 / (1.0 + x) for x in es]
            for k in range(nk):
                sh = pl.ds(D + k * 16, 16)
                lc = tp[e, sh] + up[e, sh] + bp[e, sh]
                mp[e, pl.ds(k * 16, 16)] = gs[k] * jnp.maximum(lc, 0.0)
            return c2

        lax.fori_loop(0, CH, erow, 0)

    def scatter_async(s):
        sv[s][...] = i1v[s][...]
        pltpu.async_copy(mv[s], agg.at[sv[s]], sems[s], add=True)

    def wait_sct(s):
        pltpu.make_async_copy(mv[s], agg.at[pl.ds(0, CH)], sems[s]).wait()

    def half(s, t, first):
        wait_idx((s + 2) % 3)
        issue_dat((s + 2) % 3, t + 2)
        wait_dat(s)
        if not first:
            wait_sct(s)
        compute(s)
        scatter_async(s)
        issue_idx(s, t + 3)

    row0 = sid * RPS
    pltpu.sync_copy(z_hbm, agg.at[pl.ds(row0, RPS)])
    plsc.subcore_barrier()

    issue_idx(0, 0)
    issue_idx(1, 1)
    wait_idx(0)
    issue_dat(0, 0)
    wait_idx(1)
    issue_dat(1, 1)
    issue_idx(2, 2)

    for s in range(3):
        half(s, s, first=True)

    def body(tt, carry):
        t0 = 3 * tt
        for s in range(3):
            half(s, t0 + s, first=False)
        return carry

    lax.fori_loop(1, TPC // 3, body, 0)
    wait_dat(0)
    wait_sct(0)
    compute(0)
    sv[0][...] = i1v[0][...]
    pltpu.sync_copy(mv[0], agg.at[sv[0]], add=True)
    wait_idx(2)
    wait_dat(1)
    wait_sct(1)
    wait_sct(2)

    plsc.subcore_barrier()
    pltpu.sync_copy(agg.at[pl.ds(row0, RPS)],
                    out_hbm.at[cid, pl.ds(row0, RPS)])


def _make_sc(t_tab, u_tab, bb, i1, i2, zrows):
    mesh = plsc.VectorSubcoreMesh(core_axis_name="c", subcore_axis_name="s")
    k = functools.partial(
        pl.kernel,
        mesh=mesh,
        out_type=jax.ShapeDtypeStruct((NC, NPAD, D), jnp.float32),
        scratch_types=(
            [pltpu.VMEM((CH,), jnp.int32)] * 6
            + [pltpu.VMEM((CH, DD), jnp.float32)] * 9
            + [pltpu.VMEM((CH, D), jnp.float32)] * 3
            + [pltpu.VMEM((CH,), jnp.int32)] * 3
            + [pltpu.VMEM_SHARED((NPAD, D), jnp.float32)]
            + [pltpu.SemaphoreType.DMA] * 9
        ),
    )(_sc_body)
    return k(t_tab, u_tab, bb, i1, i2, zrows)


def _combine_body(s_ref, a_ref, o_ref):
    a = a_ref[...]
    o_ref[...] = s_ref[...] + (a[0] + a[1])[None]


def _make_combine(sites, agg):
    bn = 1000
    return pl.pallas_call(
        _combine_body,
        grid=(N // bn,),
        in_specs=[
            pl.BlockSpec((1, bn, D), lambda i: (0, i, 0)),
            pl.BlockSpec((NC, bn, D), lambda i: (0, i, 0)),
        ],
        out_specs=pl.BlockSpec((1, bn, D), lambda i: (0, i, 0)),
        out_shape=jax.ShapeDtypeStruct((1, N, D), jnp.float32),
    )(sites, agg)


def kernel(sites, bonds, indices1, indices2, W_sig, b_sig, W_soft, b_soft):
    sites2d = sites[0]
    bonds2d = bonds[0]
    w_t = jnp.concatenate([W_sig[:, :D], W_soft[:, :D]], axis=0)
    w_u = jnp.concatenate([W_sig[:, D:2 * D], W_soft[:, D:2 * D]], axis=0)
    w_b = jnp.concatenate([W_sig[:, 2 * D:], W_soft[:, 2 * D:]], axis=0)
    bias = jnp.concatenate([b_sig, b_soft])[None, :]
    zrows = jnp.zeros((RPS, D), jnp.float32)

    t_tab, u_tab = _make_tables(sites2d, w_t, w_u)
    bb = _make_bond(bonds2d, w_b, bias)
    agg = _make_sc(t_tab, u_tab, bb, indices1, indices2, zrows)
    return _make_combine(sites, agg)


# speedup vs baseline: 2.3045x; 1.0189x over previous
import functools

import jax
import jax.numpy as jnp
from jax import lax
from jax.experimental import pallas as pl
from jax.experimental.pallas import tpu as pltpu
from jax.experimental.pallas import tpu_sc as plsc

N = 10000
E = 320000
D = 128
DB = 16
DD = 2 * D

NC = 2
NS = 16
NW = NC * NS
EPW = E // NW
CH = 16
TPC = EPW // CH
NPAD = 10240
RPS = NPAD // NS


def _tables_body(s_ref, wt_ref, wu_ref, t_ref, u_ref):
    s = s_ref[...]
    dn = (((1,), (1,)), ((), ()))
    t_ref[...] = lax.dot_general(s, wt_ref[...], dn,
                                 preferred_element_type=jnp.float32)
    u_ref[...] = lax.dot_general(s, wu_ref[...], dn,
                                 preferred_element_type=jnp.float32)


def _make_tables(sites2d, w_t, w_u):
    bn = 1000
    return pl.pallas_call(
        _tables_body,
        grid=(N // bn,),
        in_specs=[
            pl.BlockSpec((bn, D), lambda i: (i, 0)),
            pl.BlockSpec((DD, D), lambda i: (0, 0)),
            pl.BlockSpec((DD, D), lambda i: (0, 0)),
        ],
        out_specs=[
            pl.BlockSpec((bn, DD), lambda i: (i, 0)),
            pl.BlockSpec((bn, DD), lambda i: (i, 0)),
        ],
        out_shape=[
            jax.ShapeDtypeStruct((N, DD), jnp.float32),
            jax.ShapeDtypeStruct((N, DD), jnp.float32),
        ],
    )(sites2d, w_t, w_u)


def _bond_body(b_ref, wb_ref, bias_ref, bb_ref):
    dn = (((1,), (1,)), ((), ()))
    bb_ref[...] = lax.dot_general(b_ref[...], wb_ref[...], dn,
                                  preferred_element_type=jnp.float32) + bias_ref[...]


def _make_bond(bonds2d, w_b, bias):
    be = 4000
    return pl.pallas_call(
        _bond_body,
        grid=(E // be,),
        in_specs=[
            pl.BlockSpec((be, DB), lambda i: (i, 0)),
            pl.BlockSpec((DD, DB), lambda i: (0, 0)),
            pl.BlockSpec((1, DD), lambda i: (0, 0)),
        ],
        out_specs=pl.BlockSpec((be, DD), lambda i: (i, 0)),
        out_shape=jax.ShapeDtypeStruct((E, DD), jnp.float32),
    )(bonds2d, w_b, bias)


def _sc_body(t_hbm, u_hbm, bb_hbm, i1_hbm, i2_hbm, z_hbm, out_hbm,
             i1v0, i1v1, i1v2, i2v0, i2v1, i2v2, tv0, tv1, tv2,
             uv0, uv1, uv2, bbv0, bbv1, bbv2, mv0, mv1, mv2,
             sv0, sv1, sv2, agg,
             semi0, semi1, semi2, semd0, semd1, semd2,
             sems0, sems1, sems2):
    cid = lax.axis_index("c")
    sid = lax.axis_index("s")
    wid = sid * NC + cid
    i1v = [i1v0, i1v1, i1v2]
    i2v = [i2v0, i2v1, i2v2]
    tv = [tv0, tv1, tv2]
    uv = [uv0, uv1, uv2]
    bbv = [bbv0, bbv1, bbv2]
    mv = [mv0, mv1, mv2]
    sv = [sv0, sv1, sv2]
    semi = [semi0, semi1, semi2]
    semd = [semd0, semd1, semd2]
    sems = [sems0, sems1, sems2]

    def cbase(t):
        t = jnp.minimum(t, TPC - 1)
        return pl.multiple_of(wid * EPW + t * CH, 8)

    def issue_idx(p, t):
        base = cbase(t)
        pltpu.async_copy(i1_hbm.at[pl.ds(base, CH)], i1v[p], semi[p])
        pltpu.async_copy(i2_hbm.at[pl.ds(base, CH)], i2v[p], semi[p])

    def wait_idx(p):
        pltpu.make_async_copy(i1_hbm.at[pl.ds(0, CH)], i1v[p], semi[p]).wait()
        pltpu.make_async_copy(i2_hbm.at[pl.ds(0, CH)], i2v[p], semi[p]).wait()

    def issue_dat(p, t):
        base = cbase(t)
        pltpu.async_copy(t_hbm.at[i1v[p]], tv[p], semd[p])
        pltpu.async_copy(u_hbm.at[i2v[p]], uv[p], semd[p])
        pltpu.async_copy(bb_hbm.at[pl.ds(base, CH)], bbv[p], semd[p])

    def wait_dat(p):
        pltpu.make_async_copy(t_hbm.at[pl.ds(0, CH)], tv[p], semd[p]).wait()
        pltpu.make_async_copy(u_hbm.at[pl.ds(0, CH)], uv[p], semd[p]).wait()
        pltpu.make_async_copy(bb_hbm.at[pl.ds(0, CH)], bbv[p], semd[p]).wait()

    def compute(p):
        tp, up, bp = tv[p], uv[p], bbv[p]
        mp = mv[p]

        def erow(e2, c2):
            nk = D // 16
            e = e2 * 2
            es = []
            for ee in range(2):
                for k in range(nk):
                    sl = pl.ds(k * 16, 16)
                    es.append(jnp.exp(-(tp[e + ee, sl] + up[e + ee, sl]
                                        + bp[e + ee, sl])))
            gs = [1.0 / (1.0 + x) for x in es]
            for ee in range(2):
                for k in range(nk):
                    sh = pl.ds(D + k * 16, 16)
                    lc = tp[e + ee, sh] + up[e + ee, sh] + bp[e + ee, sh]
                    mp[e + ee, pl.ds(k * 16, 16)] = (
                        gs[ee * nk + k] * jnp.maximum(lc, 0.0))
            return c2

        lax.fori_loop(0, CH // 2, erow, 0)

    def scatter_async(s):
        sv[s][...] = i1v[s][...]
        pltpu.async_copy(mv[s], agg.at[sv[s]], sems[s], add=True)

    def wait_sct(s):
        pltpu.make_async_copy(mv[s], agg.at[pl.ds(0, CH)], sems[s]).wait()

    def half(s, t, first):
        wait_idx((s + 2) % 3)
        issue_dat((s + 2) % 3, t + 2)
        wait_dat(s)
        if not first:
            wait_sct(s)
        compute(s)
        scatter_async(s)
        issue_idx(s, t + 3)

    row0 = sid * RPS
    pltpu.sync_copy(z_hbm, agg.at[pl.ds(row0, RPS)])
    plsc.subcore_barrier()

    issue_idx(0, 0)
    issue_idx(1, 1)
    wait_idx(0)
    issue_dat(0, 0)
    wait_idx(1)
    issue_dat(1, 1)
    issue_idx(2, 2)

    for s in range(3):
        half(s, s, first=True)

    def body(tt, carry):
        t0 = 3 * tt
        for s in range(3):
            half(s, t0 + s, first=False)
        return carry

    lax.fori_loop(1, TPC // 3, body, 0)
    wait_dat(0)
    wait_sct(0)
    compute(0)
    sv[0][...] = i1v[0][...]
    pltpu.sync_copy(mv[0], agg.at[sv[0]], add=True)
    wait_idx(2)
    wait_dat(1)
    wait_sct(1)
    wait_sct(2)

    plsc.subcore_barrier()
    pltpu.sync_copy(agg.at[pl.ds(row0, RPS)],
                    out_hbm.at[cid, pl.ds(row0, RPS)])


def _make_sc(t_tab, u_tab, bb, i1, i2, zrows):
    mesh = plsc.VectorSubcoreMesh(core_axis_name="c", subcore_axis_name="s")
    k = functools.partial(
        pl.kernel,
        mesh=mesh,
        out_type=jax.ShapeDtypeStruct((NC, NPAD, D), jnp.float32),
        scratch_types=(
            [pltpu.VMEM((CH,), jnp.int32)] * 6
            + [pltpu.VMEM((CH, DD), jnp.float32)] * 9
            + [pltpu.VMEM((CH, D), jnp.float32)] * 3
            + [pltpu.VMEM((CH,), jnp.int32)] * 3
            + [pltpu.VMEM_SHARED((NPAD, D), jnp.float32)]
            + [pltpu.SemaphoreType.DMA] * 9
        ),
    )(_sc_body)
    return k(t_tab, u_tab, bb, i1, i2, zrows)


def _combine_body(s_ref, a_ref, o_ref):
    a = a_ref[...]
    o_ref[...] = s_ref[...] + (a[0] + a[1])[None]


def _make_combine(sites, agg):
    bn = 1000
    return pl.pallas_call(
        _combine_body,
        grid=(N // bn,),
        in_specs=[
            pl.BlockSpec((1, bn, D), lambda i: (0, i, 0)),
            pl.BlockSpec((NC, bn, D), lambda i: (0, i, 0)),
        ],
        out_specs=pl.BlockSpec((1, bn, D), lambda i: (0, i, 0)),
        out_shape=jax.ShapeDtypeStruct((1, N, D), jnp.float32),
    )(sites, agg)


def kernel(sites, bonds, indices1, indices2, W_sig, b_sig, W_soft, b_soft):
    sites2d = sites[0]
    bonds2d = bonds[0]
    w_t = jnp.concatenate([W_sig[:, :D], W_soft[:, :D]], axis=0)
    w_u = jnp.concatenate([W_sig[:, D:2 * D], W_soft[:, D:2 * D]], axis=0)
    w_b = jnp.concatenate([W_sig[:, 2 * D:], W_soft[:, 2 * D:]], axis=0)
    bias = jnp.concatenate([b_sig, b_soft])[None, :]
    zrows = jnp.zeros((RPS, D), jnp.float32)

    t_tab, u_tab = _make_tables(sites2d, w_t, w_u)
    bb = _make_bond(bonds2d, w_b, bias)
    agg = _make_sc(t_tab, u_tab, bb, indices1, indices2, zrows)
    return _make_combine(sites, agg)
